# Initial kernel scaffold; baseline (speedup 1.0000x reference)
import jax
import jax.numpy as jnp
from jax.experimental import pallas as pl


def kernel(table):
    raise NotImplementedError("write your pallas kernel here")


# speedup vs baseline: 3.2580x; 3.2580x over previous
import jax
import jax.numpy as jnp
from jax.experimental import pallas as pl

_N = 100000
_H = 128
_BLOCK = 5000


def _copy_body(in_ref, out_ref):
    out_ref[...] = in_ref[...]


def kernel(table):
    grid = (_N // _BLOCK,)
    return pl.pallas_call(
        _copy_body,
        grid=grid,
        in_specs=[pl.BlockSpec((_BLOCK, _H), lambda i: (i, 0))],
        out_specs=pl.BlockSpec((_BLOCK, _H), lambda i: (i, 0)),
        out_shape=jax.ShapeDtypeStruct((_N, _H), table.dtype),
    )(table)


# speedup vs baseline: 3.4747x; 1.0665x over previous
import jax
import jax.numpy as jnp
from jax.experimental import pallas as pl

_N = 100000
_H = 128
_BLOCK = 10000


def _copy_body(in_ref, out_ref):
    out_ref[...] = in_ref[...]


def kernel(table):
    grid = (_N // _BLOCK,)
    return pl.pallas_call(
        _copy_body,
        grid=grid,
        in_specs=[pl.BlockSpec((_BLOCK, _H), lambda i: (i, 0))],
        out_specs=pl.BlockSpec((_BLOCK, _H), lambda i: (i, 0)),
        out_shape=jax.ShapeDtypeStruct((_N, _H), table.dtype),
    )(table)
